# baseline (speedup 1.0000x reference)
import jax
import jax.numpy as jnp
from jax.experimental import pallas as pl

N_NODES = 50000
E_EDGES = 800000
H1 = 32
H2 = 32
PACK = 4
NPAD = 25088


def _leaky(x):
    return jnp.where(x > 0, x, 0.2 * x)


def _bd(w):
    return jnp.kron(jnp.eye(PACK, dtype=w.dtype), w)


def _bt(b):
    return jnp.tile(b, PACK)[None, :]


def _mlp_out_body(gs, gd, ef, w1s, w1d, w1e, b1, w2, b2, w3, b3, w4, b4,
                  w5, b5, out):
    x = gs[...] @ w1s[...] + gd[...] @ w1d[...] + ef[...] @ w1e[...] + b1[...]
    x = _leaky(x)
    x = _leaky(x @ w2[...] + b2[...])
    x = _leaky(x @ w3[...] + b3[...])
    x = _leaky(x @ w4[...] + b4[...])
    out[...] = x @ w5[...] + b5[...]


def _edge_mlp_out(gs, gd, ef, ps):
    e4 = E_EDGES // PACK
    (w1, b1), (w2, b2), (w3, b3), (w4, b4), (w5, b5) = ps
    args = (gs.reshape(e4, 128), gd.reshape(e4, 128), ef.reshape(e4, 64),
            _bd(w1[:32]), _bd(w1[32:64]), _bd(w1[64:80]), _bt(b1),
            _bd(w2), _bt(b2), _bd(w3), _bt(b3), _bd(w4), _bt(b4),
            _bd(w5), _bt(b5))
    R = 2000
    in_specs = [
        pl.BlockSpec((R, 128), lambda i: (i, 0)),
        pl.BlockSpec((R, 128), lambda i: (i, 0)),
        pl.BlockSpec((R, 64), lambda i: (i, 0)),
    ] + [pl.BlockSpec(a.shape, lambda i: (0, 0)) for a in args[3:]]
    out = pl.pallas_call(
        _mlp_out_body,
        grid=(e4 // R,),
        in_specs=in_specs,
        out_specs=pl.BlockSpec((R, 256), lambda i: (i, 0)),
        out_shape=jax.ShapeDtypeStruct((e4, 256), jnp.float32),
    )(*args)
    return out.reshape(E_EDGES, 64)


def _mlp_in_body(gs, gd, ef, w1s, w1d, w1e, b1, w2, b2, w3, b3,
                 w5k, b5k, w5a, b5a, w5b, b5b, f1o, f2o):
    x = gs[...] @ w1s[...] + gd[...] @ w1d[...] + ef[...] @ w1e[...] + b1[...]
    x = _leaky(x)
    x = _leaky(x @ w2[...] + b2[...])
    x = _leaky(x @ w3[...] + b3[...])
    k = 1.0 / (1.0 + jnp.exp(-(x @ w5k[...] + b5k[...])))
    f1o[...] = (x @ w5a[...] + b5a[...]) * k
    f2o[...] = (x @ w5b[...] + b5b[...]) * k


def _edge_mlp_in(gs, gd, ef, ps):
    e4 = E_EDGES // PACK
    (w1, b1), (w2, b2), (w3, b3), (w5, b5) = ps
    w5k = jnp.tile(w5[:, 0:1], (1, 32))
    b5k = jnp.tile(b5[0:1], 32)
    args = (gs.reshape(e4, 128), gd.reshape(e4, 128), ef.reshape(e4, 64),
            _bd(w1[:32]), _bd(w1[32:64]), _bd(w1[64:80]), _bt(b1),
            _bd(w2), _bt(b2), _bd(w3), _bt(b3),
            _bd(w5k), _bt(b5k),
            _bd(w5[:, 1:1 + H1]), _bt(b5[1:1 + H1]),
            _bd(w5[:, 1 + H1:1 + H1 + H2]), _bt(b5[1 + H1:1 + H1 + H2]))
    R = 2000
    in_specs = [
        pl.BlockSpec((R, 128), lambda i: (i, 0)),
        pl.BlockSpec((R, 128), lambda i: (i, 0)),
        pl.BlockSpec((R, 64), lambda i: (i, 0)),
    ] + [pl.BlockSpec(a.shape, lambda i: (0, 0)) for a in args[3:]]
    f1, f2 = pl.pallas_call(
        _mlp_in_body,
        grid=(e4 // R,),
        in_specs=in_specs,
        out_specs=[pl.BlockSpec((R, 128), lambda i: (i, 0)),
                   pl.BlockSpec((R, 128), lambda i: (i, 0))],
        out_shape=[jax.ShapeDtypeStruct((e4, 128), jnp.float32),
                   jax.ShapeDtypeStruct((e4, 128), jnp.float32)],
    )(*args)
    return f1.reshape(E_EDGES, 32), f2.reshape(E_EDGES, 32)


def _node_body(nfi_g, nfisum_g, nfo_g, f1s_g, f2m_g,
               cnt_b, cnto_ob, cnti_ib, cnto_ib, validm,
               wi1a, wi1b, bi1, wi2, bi2, wi3, bi3, wi4, bi4,
               wo1a, wo1b, wo1c, bo1, wo2, bo2, wo3, bo3, wo4, bo4,
               ei_o, eo_o):
    x = nfi_g[...] @ wi1a[...] + nfisum_g[...] @ wi1b[...] + bi1[...]
    x = _leaky(x)
    x = _leaky(x @ wi2[...] + bi2[...])
    x = _leaky(x @ wi3[...] + bi3[...])
    new_i = x @ wi4[...] + bi4[...]
    keep = jnp.where(cnto_ib[...] > 0.0, 0.0, 1.0)
    ei_o[...] = new_i * validm[...] * keep / jnp.maximum(cnti_ib[...], 1.0)

    cb = cnt_b[...]
    nfo1 = f1s_g[...] / jnp.maximum(cb, 1.0)
    nfo2 = jnp.where(cb > 0.0, f2m_g[...], 0.0)
    y = (nfo_g[...] @ wo1a[...] + nfo1 @ wo1b[...] + nfo2 @ wo1c[...]
         + bo1[...])
    y = _leaky(y)
    y = _leaky(y @ wo2[...] + bo2[...])
    y = _leaky(y @ wo3[...] + bo3[...])
    new_o = y @ wo4[...] + bo4[...]
    eo_o[...] = new_o * validm[...] / jnp.maximum(cnto_ob[...], 1.0)


def _node_mlps(nf_i, nfi_i, nf_o, f1s_o, f2m_o,
               cnt_b, cnto_ob, cnti_ib, cnto_ib, validm, pi, po):
    n4 = NPAD // PACK
    (wi1, bi1), (wi2, bi2), (wi3, bi3), (wi4, bi4) = pi
    (wo1, bo1), (wo2, bo2), (wo3, bo3), (wo4, bo4) = po
    args = (nf_i.reshape(n4, 128), nfi_i.reshape(n4, 256),
            nf_o.reshape(n4, 128), f1s_o.reshape(n4, 128),
            f2m_o.reshape(n4, 128),
            cnt_b.reshape(n4, 128), cnto_ob.reshape(n4, 128),
            cnti_ib.reshape(n4, 128), cnto_ib.reshape(n4, 128),
            validm.reshape(n4, 128),
            _bd(wi1[:32]), _bd(wi1[32:96]), _bt(bi1),
            _bd(wi2), _bt(bi2), _bd(wi3), _bt(bi3), _bd(wi4), _bt(bi4),
            _bd(wo1[:32]), _bd(wo1[32:64]), _bd(wo1[64:96]), _bt(bo1),
            _bd(wo2), _bt(bo2), _bd(wo3), _bt(bo3), _bd(wo4), _bt(bo4))
    ei, eo = pl.pallas_call(
        _node_body,
        grid=(1,),
        in_specs=[pl.BlockSpec(a.shape, lambda i: (0, 0)) for a in args],
        out_specs=[pl.BlockSpec((n4, 128), lambda i: (0, 0)),
                   pl.BlockSpec((n4, 128), lambda i: (0, 0))],
        out_shape=[jax.ShapeDtypeStruct((n4, 128), jnp.float32),
                   jax.ShapeDtypeStruct((n4, 128), jnp.float32)],
    )(*args)
    return ei.reshape(NPAD, 32), eo.reshape(NPAD, 32)


def kernel(nf, ef_out, ef_in, params, edge_index_out, edge_index_in,
           input_nodes, output_nodes):
    n = nf.shape[0]
    src_o, dst_o = edge_index_out[0], edge_index_out[1]
    src_i, dst_i = edge_index_in[0], edge_index_in[1]

    gso, gdo = nf[src_o], nf[dst_o]
    gsi, gdi = nf[src_i], nf[dst_i]

    efi = _edge_mlp_out(gso, gdo, ef_out, params['msg_o2i'])
    f1, f2 = _edge_mlp_in(gsi, gdi, ef_in, params['msg_i2o'])

    nfi = jax.ops.segment_sum(efi, dst_o, num_segments=n)
    ones = jnp.ones((E_EDGES,), dtype=jnp.float32)
    cnt = jax.ops.segment_sum(ones, dst_i, num_segments=n)
    f1sum = jax.ops.segment_sum(f1, dst_i, num_segments=n)
    f2max = jax.ops.segment_max(f2, dst_i, num_segments=n)
    oi = jnp.ones((input_nodes.shape[0],), dtype=jnp.float32)
    cnt_i = jax.ops.segment_sum(oi, input_nodes, num_segments=n)
    cnt_o = jax.ops.segment_sum(oi, output_nodes, num_segments=n)

    pad = NPAD - input_nodes.shape[0]
    inp_g = jnp.pad(input_nodes, (0, pad))
    out_g = jnp.pad(output_nodes, (0, pad))
    bc = lambda v: jnp.broadcast_to(v[:, None], (NPAD, 32))
    validm = bc((jnp.arange(NPAD) < input_nodes.shape[0]).astype(jnp.float32))

    ei, eo = _node_mlps(
        nf[inp_g], nfi[inp_g], nf[out_g], f1sum[out_g], f2max[out_g],
        bc(cnt[out_g]), bc(cnt_o[out_g]), bc(cnt_i[inp_g]), bc(cnt_o[inp_g]),
        validm, params['reduce_i'], params['reduce_o'])

    new_nf = jnp.zeros((n, 32), jnp.float32)
    new_nf = new_nf.at[inp_g].add(ei)
    new_nf = new_nf.at[out_g].add(eo)
    return new_nf
